# baseline (speedup 1.0000x reference)
import functools

import jax
import jax.numpy as jnp
from jax import lax
from jax.experimental import pallas as pl
from jax.experimental.pallas import tpu as pltpu
from jax.experimental.pallas import tpu_sc as plsc

_NC, _NS = 2, 16
_NW = _NC * _NS
_CH = 128


def _sc_gather(idx, table, start, nrows, d):
    rpw = nrows // _NW
    nchunk = rpw // _CH
    mesh = plsc.VectorSubcoreMesh(core_axis_name="c", subcore_axis_name="s")

    @functools.partial(
        pl.kernel,
        out_type=jax.ShapeDtypeStruct((nrows, d), jnp.float32),
        mesh=mesh,
        scratch_types=[
            pltpu.VMEM((_CH,), jnp.int32),
            pltpu.VMEM((_CH,), jnp.int32),
            pltpu.VMEM((_CH, d), jnp.float32),
            pltpu.VMEM((_CH, d), jnp.float32),
            pltpu.VMEM_SHARED((4196, d), jnp.float32),
            pltpu.SemaphoreType.DMA,
            pltpu.SemaphoreType.DMA,
            pltpu.SemaphoreType.DMA,
            pltpu.SemaphoreType.DMA,
        ],
    )
    def gather_kernel(idx_hbm, table_hbm, out_hbm, idx_v0, idx_v1,
                      rows_v0, rows_v1, tbl_sh, sg0, sg1, so0, so1):
        sid = lax.axis_index("s")

        @pl.when(sid == 0)
        def _():
            pltpu.sync_copy(table_hbm, tbl_sh)
        plsc.subcore_barrier()
        wid = lax.axis_index("s") * _NC + lax.axis_index("c")
        base = wid * rpw
        idx_bufs = (idx_v0, idx_v1)
        row_bufs = (rows_v0, rows_v1)
        g_sems = (sg0, sg1)
        o_sems = (so0, so1)
        g_descs = [None] * nchunk
        o_descs = [None] * nchunk
        for j in range(nchunk):
            bi = j % 2
            off = base + j * _CH
            if j >= 2:
                o_descs[j - 2].wait()
            pltpu.sync_copy(idx_hbm.at[pl.ds(start + off, _CH)], idx_bufs[bi])
            g_descs[j] = pltpu.async_copy(
                tbl_sh.at[idx_bufs[bi]], row_bufs[bi], g_sems[bi])
            if j >= 1:
                g_descs[j - 1].wait()
                poff = base + (j - 1) * _CH
                o_descs[j - 1] = pltpu.async_copy(
                    row_bufs[1 - bi], out_hbm.at[pl.ds(poff, _CH)],
                    o_sems[1 - bi])
        g_descs[nchunk - 1].wait()
        o_descs[nchunk - 1] = pltpu.async_copy(
            row_bufs[(nchunk - 1) % 2],
            out_hbm.at[pl.ds(base + (nchunk - 1) * _CH, _CH)],
            o_sems[(nchunk - 1) % 2])
        o_descs[nchunk - 2].wait()
        o_descs[nchunk - 1].wait()

    return gather_kernel(idx, table)


def _make_tc_body(lb, b, d, has_alias):
    tn = (((0,), (0,)), ((), ()))

    def _tc_body(s_ref, a_ref, r_ref, t_ref, ws_ref, bs_ref, wa_ref, ba_ref,
                 wr_ref, br_ref, *rest):
        o_ref = rest[1] if has_alias else rest[0]
        ws = ws_ref[...]
        wa = wa_ref[...]
        wr = wr_ref[...]
        bs_ = bs_ref[...][None, :]
        ba_ = ba_ref[...][None, :]
        br_ = br_ref[...][None, :]
        for jj in range(lb):
            t = t_ref[pl.ds(jj * b, b), :]
            se = lax.dot_general(s_ref[jj], ws, tn,
                                 preferred_element_type=jnp.float32) + bs_ + t
            ae = lax.dot_general(a_ref[jj], wa, tn,
                                 preferred_element_type=jnp.float32) + ba_ + t
            re = lax.dot_general(r_ref[jj], wr, tn,
                                 preferred_element_type=jnp.float32) + br_ + t
            o_ref[3 * jj + 0] = re
            o_ref[3 * jj + 1] = se
            o_ref[3 * jj + 2] = ae
    return _tc_body


def _tc_phase(st, at, rt, t_emb, Ws, bs, Wa, ba, Wr, br,
              l, b, d, sd, ad, t0, tc, lb, prev_out):
    grid = tc // lb
    o0 = t0 // lb
    in_specs = [
        pl.BlockSpec((lb, sd, b), lambda i, o0=o0: (i + o0, 0, 0)),
        pl.BlockSpec((lb, ad, b), lambda i, o0=o0: (i + o0, 0, 0)),
        pl.BlockSpec((lb, 1, b), lambda i, o0=o0: (i + o0, 0, 0)),
        pl.BlockSpec((lb * b, d), lambda i: (i, 0)),
        pl.BlockSpec((sd, d), lambda i: (0, 0)),
        pl.BlockSpec((d,), lambda i: (0,)),
        pl.BlockSpec((ad, d), lambda i: (0, 0)),
        pl.BlockSpec((d,), lambda i: (0,)),
        pl.BlockSpec((1, d), lambda i: (0, 0)),
        pl.BlockSpec((d,), lambda i: (0,)),
    ]
    inputs = [st, at, rt, t_emb, Ws, bs, Wa, ba, Wr, br]
    kwargs = {}
    if prev_out is not None:
        in_specs.append(pl.BlockSpec(memory_space=pltpu.MemorySpace.HBM))
        inputs.append(prev_out)
        kwargs['input_output_aliases'] = {10: 0}
    return pl.pallas_call(
        _make_tc_body(lb, b, d, prev_out is not None),
        grid=(grid,),
        in_specs=in_specs,
        out_specs=pl.BlockSpec((3 * lb, b, d), lambda i, o0=o0: (i + o0, 0, 0)),
        out_shape=jax.ShapeDtypeStruct((3 * l, b, d), jnp.float32),
        **kwargs,
    )(*inputs)


def kernel(states, actions, returns_to_go, time_steps, timestep_table,
           Ws, bs, Wa, ba, Wr, br):
    b, l, sd = states.shape
    ad = actions.shape[-1]
    d = timestep_table.shape[-1]
    rows = b * l

    st = states.transpose(1, 2, 0)
    at = actions.transpose(1, 2, 0)
    rt = returns_to_go.transpose(1, 2, 0)
    idx = time_steps.transpose(1, 0).reshape(rows).astype(jnp.int32)

    lb = 4
    phases = (16, 40, 44)
    t_embs = []
    t0 = 0
    for tc in phases:
        t_embs.append(_sc_gather(idx, timestep_table, t0 * b, tc * b, d))
        t0 += tc

    out = None
    t0 = 0
    for tc, t_emb in zip(phases, t_embs):
        out = _tc_phase(st, at, rt, t_emb, Ws, bs, Wa, ba, Wr, br,
                        l, b, d, sd, ad, t0, tc, lb, out)
        t0 += tc
    return out.transpose(1, 0, 2)


# speedup vs baseline: 1.0194x; 1.0194x over previous
import functools

import jax
import jax.numpy as jnp
from jax import lax
from jax.experimental import pallas as pl
from jax.experimental.pallas import tpu as pltpu
from jax.experimental.pallas import tpu_sc as plsc

_NC, _NS = 2, 16
_NW = _NC * _NS
_CH = 128


def _sc_gather(idx, table, start, nrows, d):
    rpw = nrows // _NW
    nchunk = rpw // _CH
    mesh = plsc.VectorSubcoreMesh(core_axis_name="c", subcore_axis_name="s")

    @functools.partial(
        pl.kernel,
        out_type=jax.ShapeDtypeStruct((nrows, d), jnp.float32),
        mesh=mesh,
        scratch_types=[
            pltpu.VMEM((_CH,), jnp.int32),
            pltpu.VMEM((_CH,), jnp.int32),
            pltpu.VMEM((_CH, d), jnp.float32),
            pltpu.VMEM((_CH, d), jnp.float32),
            pltpu.VMEM_SHARED((4196, d), jnp.float32),
            pltpu.SemaphoreType.DMA,
            pltpu.SemaphoreType.DMA,
            pltpu.SemaphoreType.DMA,
            pltpu.SemaphoreType.DMA,
        ],
    )
    def gather_kernel(idx_hbm, table_hbm, out_hbm, idx_v0, idx_v1,
                      rows_v0, rows_v1, tbl_sh, sg0, sg1, so0, so1):
        sid = lax.axis_index("s")

        @pl.when(sid == 0)
        def _():
            pltpu.sync_copy(table_hbm, tbl_sh)
        plsc.subcore_barrier()
        wid = lax.axis_index("s") * _NC + lax.axis_index("c")
        base = wid * rpw
        idx_bufs = (idx_v0, idx_v1)
        row_bufs = (rows_v0, rows_v1)
        g_sems = (sg0, sg1)
        o_sems = (so0, so1)
        g_descs = [None] * nchunk
        o_descs = [None] * nchunk
        for j in range(nchunk):
            bi = j % 2
            off = base + j * _CH
            if j >= 2:
                o_descs[j - 2].wait()
            pltpu.sync_copy(idx_hbm.at[pl.ds(start + off, _CH)], idx_bufs[bi])
            g_descs[j] = pltpu.async_copy(
                tbl_sh.at[idx_bufs[bi]], row_bufs[bi], g_sems[bi])
            if j >= 1:
                g_descs[j - 1].wait()
                poff = base + (j - 1) * _CH
                o_descs[j - 1] = pltpu.async_copy(
                    row_bufs[1 - bi], out_hbm.at[pl.ds(poff, _CH)],
                    o_sems[1 - bi])
        g_descs[nchunk - 1].wait()
        o_descs[nchunk - 1] = pltpu.async_copy(
            row_bufs[(nchunk - 1) % 2],
            out_hbm.at[pl.ds(base + (nchunk - 1) * _CH, _CH)],
            o_sems[(nchunk - 1) % 2])
        o_descs[nchunk - 2].wait()
        o_descs[nchunk - 1].wait()

    return gather_kernel(idx, table)


def _make_tc_body(lb, b, d, has_alias):
    tn = (((0,), (0,)), ((), ()))

    def _tc_body(s_ref, a_ref, r_ref, t_ref, ws_ref, bs_ref, wa_ref, ba_ref,
                 wr_ref, br_ref, *rest):
        o_ref = rest[1] if has_alias else rest[0]
        ws = ws_ref[...]
        wa = wa_ref[...]
        wr = wr_ref[...]
        bs_ = bs_ref[...][None, :]
        ba_ = ba_ref[...][None, :]
        br_ = br_ref[...][None, :]
        for jj in range(lb):
            t = t_ref[pl.ds(jj * b, b), :]
            se = lax.dot_general(s_ref[jj], ws, tn,
                                 preferred_element_type=jnp.float32) + bs_ + t
            ae = lax.dot_general(a_ref[jj], wa, tn,
                                 preferred_element_type=jnp.float32) + ba_ + t
            re = lax.dot_general(r_ref[jj], wr, tn,
                                 preferred_element_type=jnp.float32) + br_ + t
            o_ref[3 * jj + 0] = re
            o_ref[3 * jj + 1] = se
            o_ref[3 * jj + 2] = ae
    return _tc_body


def _tc_phase(st, at, rt, t_emb, Ws, bs, Wa, ba, Wr, br,
              l, b, d, sd, ad, t0, tc, lb, prev_out):
    grid = tc // lb
    o0 = t0 // lb
    in_specs = [
        pl.BlockSpec((lb, sd, b), lambda i, o0=o0: (i + o0, 0, 0)),
        pl.BlockSpec((lb, ad, b), lambda i, o0=o0: (i + o0, 0, 0)),
        pl.BlockSpec((lb, 1, b), lambda i, o0=o0: (i + o0, 0, 0)),
        pl.BlockSpec((lb * b, d), lambda i: (i, 0)),
        pl.BlockSpec((sd, d), lambda i: (0, 0)),
        pl.BlockSpec((d,), lambda i: (0,)),
        pl.BlockSpec((ad, d), lambda i: (0, 0)),
        pl.BlockSpec((d,), lambda i: (0,)),
        pl.BlockSpec((1, d), lambda i: (0, 0)),
        pl.BlockSpec((d,), lambda i: (0,)),
    ]
    inputs = [st, at, rt, t_emb, Ws, bs, Wa, ba, Wr, br]
    kwargs = {}
    if prev_out is not None:
        in_specs.append(pl.BlockSpec(memory_space=pltpu.MemorySpace.HBM))
        inputs.append(prev_out)
        kwargs['input_output_aliases'] = {10: 0}
    return pl.pallas_call(
        _make_tc_body(lb, b, d, prev_out is not None),
        grid=(grid,),
        in_specs=in_specs,
        out_specs=pl.BlockSpec((3 * lb, b, d), lambda i, o0=o0: (i + o0, 0, 0)),
        out_shape=jax.ShapeDtypeStruct((3 * l, b, d), jnp.float32),
        **kwargs,
    )(*inputs)


def kernel(states, actions, returns_to_go, time_steps, timestep_table,
           Ws, bs, Wa, ba, Wr, br):
    b, l, sd = states.shape
    ad = actions.shape[-1]
    d = timestep_table.shape[-1]
    rows = b * l

    st = states.transpose(1, 2, 0)
    at = actions.transpose(1, 2, 0)
    rt = returns_to_go.transpose(1, 2, 0)
    idx = time_steps.transpose(1, 0).reshape(rows).astype(jnp.int32)

    lb = 4
    phases = (24, 36, 40)
    t_embs = []
    t0 = 0
    for tc in phases:
        t_embs.append(_sc_gather(idx, timestep_table, t0 * b, tc * b, d))
        t0 += tc

    out = None
    t0 = 0
    for tc, t_emb in zip(phases, t_embs):
        out = _tc_phase(st, at, rt, t_emb, Ws, bs, Wa, ba, Wr, br,
                        l, b, d, sd, ad, t0, tc, lb, out)
        t0 += tc
    return out.transpose(1, 0, 2)
